# baseline (speedup 1.0000x reference)
import jax
import jax.numpy as jnp
from jax import lax
from jax.experimental import pallas as pl
from jax.experimental.pallas import tpu as pltpu
from jax.experimental.pallas import tpu_sc as plsc

N_USERS = 25000
N_NODES = 50000
N_EDGES = 1600000
D = 32
DH = 16
BATCH = 4096

NC = 2
NS = 16

N_PAD = 50176
ROWS_PER_TILE = N_PAD // NS
E_PAD = 1638400
E_ROWS = E_PAD // 128
ROWS_PER_TILE_E = E_ROWS // NS
CHUNK_ROWS = 5
N_CHUNKS = ROWS_PER_TILE_E // CHUNK_ROWS

B3 = 3 * BATCH
BPT = B3 // NS

_mesh = plsc.VectorSubcoreMesh(core_axis_name="c", subcore_axis_name="s")
_sc_params = pltpu.CompilerParams(use_tc_tiling_on_sc=False)


def _edge_pipeline(s, tbl, acc, src_hbm, dst_hbm, w_hbm,
                   rb, sb, db, wb, esem, gsem, ssem):
    base_e = s * ROWS_PER_TILE_E * 128
    ce = CHUNK_ROWS * 128

    def issue_edge_loads(g, e):
        e0 = base_e + g * ce
        pltpu.async_copy(src_hbm.at[pl.ds(e0, ce)], sb[e], esem[e])
        pltpu.async_copy(dst_hbm.at[pl.ds(e0, ce)], db[e], esem[e])
        pltpu.async_copy(w_hbm.at[pl.ds(e0, ce)], wb[e], esem[e])

    def wait_edge_loads(g, e):
        e0 = base_e + g * ce
        pltpu.make_async_copy(src_hbm.at[pl.ds(e0, ce)], sb[e], esem[e]).wait()
        pltpu.make_async_copy(dst_hbm.at[pl.ds(e0, ce)], db[e], esem[e]).wait()
        pltpu.make_async_copy(w_hbm.at[pl.ds(e0, ce)], wb[e], esem[e]).wait()

    half = CHUNK_ROWS * 128 // 2

    def issue_gathers(b, e):
        pltpu.async_copy(tbl.at[sb[e].at[pl.ds(0, half)]],
                         rb[b].at[pl.ds(0, half)], gsem[b])
        pltpu.async_copy(tbl.at[sb[e].at[pl.ds(half, half)]],
                         rb[b].at[pl.ds(half, half)], gsem[b])

    def wait_gathers(b, e):
        pltpu.make_async_copy(tbl.at[sb[e].at[pl.ds(0, half)]],
                              rb[b].at[pl.ds(0, half)], gsem[b]).wait()
        pltpu.make_async_copy(tbl.at[sb[e].at[pl.ds(half, half)]],
                              rb[b].at[pl.ds(half, half)], gsem[b]).wait()

    def issue_scatters(b, e):
        pltpu.async_copy(rb[b].at[pl.ds(0, half)],
                         acc.at[db[e].at[pl.ds(0, half)]], ssem[b], add=True)
        pltpu.async_copy(rb[b].at[pl.ds(half, half)],
                         acc.at[db[e].at[pl.ds(half, half)]], ssem[b], add=True)

    def wait_scatters(b, e):
        pltpu.make_async_copy(rb[b].at[pl.ds(0, half)],
                              acc.at[db[e].at[pl.ds(0, half)]], ssem[b]).wait()
        pltpu.make_async_copy(rb[b].at[pl.ds(half, half)],
                              acc.at[db[e].at[pl.ds(half, half)]], ssem[b]).wait()

    lane_idx = [jnp.full((16,), k, jnp.int32) for k in range(16)]

    def multiply(b, e):
        rbb = rb[b]
        wbb = wb[e]

        @plsc.parallel_loop(0, CHUNK_ROWS * 8, step=1, unroll=2)
        def mul_body(t):
            wvec = wbb[pl.ds(t * 16, 16)]
            e0 = t * 16
            for k in range(16):
                wv = wvec.at[lane_idx[k]].get(mode="promise_in_bounds")
                rbb[e0 + k, pl.ds(0, 16)] = rbb[e0 + k, pl.ds(0, 16)] * wv

    def chunk_step(g, u):
        b = u % 2
        nb = 1 - b

        @pl.when(g >= 1)
        def _():
            wait_scatters(nb, (u + 3) % 4)

        @pl.when(g + 2 < N_CHUNKS)
        def _():
            issue_edge_loads(g + 2, (u + 2) % 4)

        @pl.when(g + 1 < N_CHUNKS)
        def _():
            wait_edge_loads(g + 1, (u + 1) % 4)
            issue_gathers(nb, (u + 1) % 4)

        wait_gathers(b, u)
        multiply(b, u)
        issue_scatters(b, u)

    issue_edge_loads(0, 0)
    issue_edge_loads(1, 1)
    wait_edge_loads(0, 0)
    issue_gathers(0, 0)

    def loop_body(gg, _):
        for u in range(4):
            chunk_step(gg * 4 + u, u)
        return 0

    lax.fori_loop(0, N_CHUNKS // 4, loop_body, 0)
    wait_scatters(1, 3)
    plsc.subcore_barrier()


def _mega_body(t_lo, t_hi, src_hbm, dst_hbm, w_hbm, idx_hbm, out_lo, out_hi,
               ta, tb, rb0, rb1, sb0, sb1, sb2, sb3, db0, db1, db2, db3,
               wb0, wb1, wb2, wb3, ib640, ib128,
               esem0, esem1, esem2, esem3, gsem0, gsem1, ssem0, ssem1):
    c = lax.axis_index("c")
    s = lax.axis_index("s")
    rb = (rb0, rb1)
    sb = (sb0, sb1, sb2, sb3)
    db = (db0, db1, db2, db3)
    wb = (wb0, wb1, wb2, wb3)
    esem = (esem0, esem1, esem2, esem3)
    gsem = (gsem0, gsem1)
    ssem = (ssem0, ssem1)
    r0 = s * ROWS_PER_TILE
    zero16 = jnp.zeros((16,), jnp.float32)
    zrows = rb0.shape[0]

    def zero_fill_rb0():
        def zfill(i, _):
            rb0[i, pl.ds(0, 16)] = zero16
            return 0

        lax.fori_loop(0, zrows, zfill, 0)

    def zero_acc(accr):
        zero_fill_rb0()
        nfull = ROWS_PER_TILE // zrows
        for r in range(nfull):
            pltpu.async_copy(rb0, accr.at[pl.ds(r0 + r * zrows, zrows)], esem0)
        rem = ROWS_PER_TILE - nfull * zrows
        pltpu.async_copy(rb0.at[pl.ds(0, rem)],
                         accr.at[pl.ds(r0 + nfull * zrows, rem)], esem0)
        for r in range(nfull):
            pltpu.make_async_copy(rb0, accr.at[pl.ds(r0 + r * zrows, zrows)],
                                  esem0).wait()
        pltpu.make_async_copy(rb0.at[pl.ds(0, rem)],
                              accr.at[pl.ds(r0 + nfull * zrows, rem)], esem0).wait()

    def batch_gather(tab, layer):
        d0 = pltpu.async_copy(tab.at[ib640], rb0, gsem0)
        d1 = pltpu.async_copy(tab.at[ib128], rb1.at[pl.ds(0, 128)], gsem1)
        d0.wait()
        d1.wait()
        base = s * BPT

        @pl.when(c == 0)
        def _():
            pltpu.sync_copy(rb0, out_lo.at[layer, pl.ds(base, 640)])
            pltpu.sync_copy(rb1.at[pl.ds(0, 128)],
                            out_lo.at[layer, pl.ds(base + 640, 128)])

        @pl.when(c == 1)
        def _():
            pltpu.sync_copy(rb0, out_hi.at[layer, pl.ds(base, 640)])
            pltpu.sync_copy(rb1.at[pl.ds(0, 128)],
                            out_hi.at[layer, pl.ds(base + 640, 128)])

    def pipeline(tab, accr):
        _edge_pipeline(s, tab, accr, src_hbm, dst_hbm, w_hbm,
                       rb, sb, db, wb, esem, gsem, ssem)

    @pl.when(c == 0)
    def _():
        pltpu.async_copy(t_lo.at[pl.ds(r0, ROWS_PER_TILE)],
                         ta.at[pl.ds(r0, ROWS_PER_TILE)], ssem0)

    @pl.when(c == 1)
    def _():
        pltpu.async_copy(t_hi.at[pl.ds(r0, ROWS_PER_TILE)],
                         ta.at[pl.ds(r0, ROWS_PER_TILE)], ssem0)

    pltpu.sync_copy(idx_hbm.at[pl.ds(s * BPT, 640)], ib640)
    pltpu.sync_copy(idx_hbm.at[pl.ds(s * BPT + 640, 128)], ib128)
    zero_acc(tb)

    @pl.when(c == 0)
    def _():
        pltpu.make_async_copy(t_lo.at[pl.ds(r0, ROWS_PER_TILE)],
                              ta.at[pl.ds(r0, ROWS_PER_TILE)], ssem0).wait()

    @pl.when(c == 1)
    def _():
        pltpu.make_async_copy(t_hi.at[pl.ds(r0, ROWS_PER_TILE)],
                              ta.at[pl.ds(r0, ROWS_PER_TILE)], ssem0).wait()

    plsc.subcore_barrier()

    batch_gather(ta, 0)
    pipeline(ta, tb)
    batch_gather(tb, 1)
    zero_acc(ta)
    plsc.subcore_barrier()
    pipeline(tb, ta)
    batch_gather(ta, 2)
    zero_acc(tb)
    plsc.subcore_barrier()
    pipeline(ta, tb)
    batch_gather(tb, 3)


_mega = pl.kernel(
    _mega_body,
    out_type=(jax.ShapeDtypeStruct((4, B3, DH), jnp.float32),
              jax.ShapeDtypeStruct((4, B3, DH), jnp.float32)),
    mesh=_mesh,
    compiler_params=_sc_params,
    scratch_types=[
        pltpu.VMEM_SHARED((N_PAD, DH), jnp.float32),
        pltpu.VMEM_SHARED((N_PAD, DH), jnp.float32),
        pltpu.VMEM((CHUNK_ROWS * 128, DH), jnp.float32),
        pltpu.VMEM((CHUNK_ROWS * 128, DH), jnp.float32),
        pltpu.VMEM((CHUNK_ROWS * 128,), jnp.int32),
        pltpu.VMEM((CHUNK_ROWS * 128,), jnp.int32),
        pltpu.VMEM((CHUNK_ROWS * 128,), jnp.int32),
        pltpu.VMEM((CHUNK_ROWS * 128,), jnp.int32),
        pltpu.VMEM((CHUNK_ROWS * 128,), jnp.int32),
        pltpu.VMEM((CHUNK_ROWS * 128,), jnp.int32),
        pltpu.VMEM((CHUNK_ROWS * 128,), jnp.int32),
        pltpu.VMEM((CHUNK_ROWS * 128,), jnp.int32),
        pltpu.VMEM((CHUNK_ROWS * 128,), jnp.float32),
        pltpu.VMEM((CHUNK_ROWS * 128,), jnp.float32),
        pltpu.VMEM((CHUNK_ROWS * 128,), jnp.float32),
        pltpu.VMEM((CHUNK_ROWS * 128,), jnp.float32),
        pltpu.VMEM((640,), jnp.int32),
        pltpu.VMEM((128,), jnp.int32),
        pltpu.SemaphoreType.DMA,
        pltpu.SemaphoreType.DMA,
        pltpu.SemaphoreType.DMA,
        pltpu.SemaphoreType.DMA,
        pltpu.SemaphoreType.DMA,
        pltpu.SemaphoreType.DMA,
        pltpu.SemaphoreType.DMA,
        pltpu.SemaphoreType.DMA,
    ],
)


_BR = 1024


def _loss_body(ul, pl_, nl, uh, ph, nh, o_ref):
    i = pl.program_id(0)

    def avg(r):
        x = r[...]
        return (x[0] + x[1] + x[2] + x[3]) * 0.25

    ue = jnp.concatenate([avg(ul), avg(uh)], axis=1)
    pe = jnp.concatenate([avg(pl_), avg(ph)], axis=1)
    ne = jnp.concatenate([avg(nl), avg(nh)], axis=1)
    ps = jnp.sum(ue * pe, axis=1, keepdims=True)
    ns = jnp.sum(ue * ne, axis=1, keepdims=True)
    x = ns - ps
    sp = jnp.maximum(x, 0.0) + jnp.log1p(jnp.exp(-jnp.abs(x)))
    reg = (jnp.sum(ul[0] ** 2) + jnp.sum(uh[0] ** 2)
           + jnp.sum(pl_[0] ** 2) + jnp.sum(ph[0] ** 2)
           + jnp.sum(nl[0] ** 2) + jnp.sum(nh[0] ** 2))
    part = jnp.sum(sp) / float(BATCH) + (1e-4 * 0.5 / float(BATCH)) * reg

    @pl.when(i == 0)
    def _():
        o_ref[...] = jnp.zeros((1, 1), jnp.float32)

    o_ref[...] = o_ref[...] + jnp.full((1, 1), part, jnp.float32)


def _loss(blo, bhi):
    nb = BATCH // _BR
    bs = (4, _BR, DH)
    specs = []
    for off in (0, nb, 2 * nb):
        specs.append(pl.BlockSpec(bs, lambda i, off=off: (0, off + i, 0)))
    return pl.pallas_call(
        _loss_body,
        grid=(nb,),
        in_specs=[specs[0], specs[1], specs[2]] * 2,
        out_specs=pl.BlockSpec((1, 1), lambda i: (0, 0)),
        out_shape=jax.ShapeDtypeStruct((1, 1), jnp.float32),
    )(blo, blo, blo, bhi, bhi, bhi)


def kernel(user_emb, item_emb, edge_weight, edge_index, users, pos, neg):
    t0 = jnp.concatenate(
        [user_emb, item_emb,
         jnp.zeros((N_PAD - N_NODES, D), jnp.float32)], axis=0)
    t0l, t0h = t0[:, :DH], t0[:, DH:]
    pad = E_PAD - N_EDGES
    src = jnp.concatenate([edge_index[0], jnp.zeros((pad,), jnp.int32)])
    dst = jnp.concatenate([edge_index[1], jnp.zeros((pad,), jnp.int32)])
    w = jnp.concatenate([edge_weight, jnp.zeros((pad,), jnp.float32)])
    idx = jnp.concatenate([users, pos + N_USERS, neg + N_USERS])

    blo, bhi = _mega(t0l, t0h, src, dst, w, idx)
    out = _loss(blo, bhi)
    return out[0, 0]


# speedup vs baseline: 1.0702x; 1.0702x over previous
import jax
import jax.numpy as jnp
from jax import lax
from jax.experimental import pallas as pl
from jax.experimental.pallas import tpu as pltpu
from jax.experimental.pallas import tpu_sc as plsc

N_USERS = 25000
N_NODES = 50000
N_EDGES = 1600000
D = 32
DH = 16
BATCH = 4096

NC = 2
NS = 16

N_PAD = 50176
ROWS_PER_TILE = N_PAD // NS
E_PAD = 1638400
E_ROWS = E_PAD // 128
ROWS_PER_TILE_E = E_ROWS // NS
CHUNK_ROWS = 5
N_CHUNKS = ROWS_PER_TILE_E // CHUNK_ROWS

B3 = 3 * BATCH
BPT = B3 // NS

_mesh = plsc.VectorSubcoreMesh(core_axis_name="c", subcore_axis_name="s")
_sc_params = pltpu.CompilerParams(use_tc_tiling_on_sc=False)


def _edge_pipeline(s, tbl, acc, src_hbm, dst_hbm, w_hbm,
                   rb, sb, db, wb, esem, gsem, ssem):
    base_e = s * ROWS_PER_TILE_E * 128
    ce = CHUNK_ROWS * 128

    def issue_edge_loads(g, e):
        e0 = base_e + g * ce
        pltpu.async_copy(src_hbm.at[pl.ds(e0, ce)], sb[e], esem[e])
        pltpu.async_copy(dst_hbm.at[pl.ds(e0, ce)], db[e], esem[e])
        pltpu.async_copy(w_hbm.at[pl.ds(e0, ce)], wb[e], esem[e])

    def wait_edge_loads(g, e):
        e0 = base_e + g * ce
        pltpu.make_async_copy(src_hbm.at[pl.ds(e0, ce)], sb[e], esem[e]).wait()
        pltpu.make_async_copy(dst_hbm.at[pl.ds(e0, ce)], db[e], esem[e]).wait()
        pltpu.make_async_copy(w_hbm.at[pl.ds(e0, ce)], wb[e], esem[e]).wait()

    def issue_gathers(b, e):
        pltpu.async_copy(tbl.at[sb[e]], rb[b], gsem[b])

    def wait_gathers(b, e):
        pltpu.make_async_copy(tbl.at[sb[e]], rb[b], gsem[b]).wait()

    def issue_scatters(b, e):
        pltpu.async_copy(rb[b], acc.at[db[e]], ssem[b], add=True)

    def wait_scatters(b, e):
        pltpu.make_async_copy(rb[b], acc.at[db[e]], ssem[b]).wait()

    lane_idx = [jnp.full((16,), k, jnp.int32) for k in range(16)]

    def multiply(b, e):
        rbb = rb[b]
        wbb = wb[e]

        def mul_body(t, _):
            wvec = wbb[pl.ds(t * 16, 16)]
            e0 = t * 16
            for k in range(16):
                wv = wvec.at[lane_idx[k]].get(mode="promise_in_bounds")
                rbb[e0 + k, pl.ds(0, 16)] = rbb[e0 + k, pl.ds(0, 16)] * wv
            return 0

        lax.fori_loop(0, CHUNK_ROWS * 8, mul_body, 0)

    def chunk_step(g, u):
        b = u % 2
        nb = 1 - b

        @pl.when(g >= 1)
        def _():
            wait_scatters(nb, (u + 3) % 4)

        @pl.when(g + 2 < N_CHUNKS)
        def _():
            issue_edge_loads(g + 2, (u + 2) % 4)

        @pl.when(g + 1 < N_CHUNKS)
        def _():
            wait_edge_loads(g + 1, (u + 1) % 4)
            issue_gathers(nb, (u + 1) % 4)

        wait_gathers(b, u)
        multiply(b, u)
        issue_scatters(b, u)

    issue_edge_loads(0, 0)
    issue_edge_loads(1, 1)
    wait_edge_loads(0, 0)
    issue_gathers(0, 0)

    def loop_body(gg, _):
        for u in range(4):
            chunk_step(gg * 4 + u, u)
        return 0

    lax.fori_loop(0, N_CHUNKS // 4, loop_body, 0)
    wait_scatters(1, 3)
    plsc.subcore_barrier()


def _mega_body(t_lo, t_hi, src_hbm, dst_hbm, w_hbm, idx_hbm, out_lo, out_hi,
               ta, tb, rb0, rb1, sb0, sb1, sb2, sb3, db0, db1, db2, db3,
               wb0, wb1, wb2, wb3, ib640, ib128,
               esem0, esem1, esem2, esem3, gsem0, gsem1, ssem0, ssem1):
    c = lax.axis_index("c")
    s = lax.axis_index("s")
    rb = (rb0, rb1)
    sb = (sb0, sb1, sb2, sb3)
    db = (db0, db1, db2, db3)
    wb = (wb0, wb1, wb2, wb3)
    esem = (esem0, esem1, esem2, esem3)
    gsem = (gsem0, gsem1)
    ssem = (ssem0, ssem1)
    r0 = s * ROWS_PER_TILE
    zero16 = jnp.zeros((16,), jnp.float32)
    zrows = rb0.shape[0]

    def zero_fill_rb0():
        def zfill(i, _):
            rb0[i, pl.ds(0, 16)] = zero16
            return 0

        lax.fori_loop(0, zrows, zfill, 0)

    def zero_acc(accr):
        zero_fill_rb0()
        nfull = ROWS_PER_TILE // zrows
        for r in range(nfull):
            pltpu.async_copy(rb0, accr.at[pl.ds(r0 + r * zrows, zrows)], esem0)
        rem = ROWS_PER_TILE - nfull * zrows
        pltpu.async_copy(rb0.at[pl.ds(0, rem)],
                         accr.at[pl.ds(r0 + nfull * zrows, rem)], esem0)
        for r in range(nfull):
            pltpu.make_async_copy(rb0, accr.at[pl.ds(r0 + r * zrows, zrows)],
                                  esem0).wait()
        pltpu.make_async_copy(rb0.at[pl.ds(0, rem)],
                              accr.at[pl.ds(r0 + nfull * zrows, rem)], esem0).wait()

    def batch_gather(tab, layer):
        d0 = pltpu.async_copy(tab.at[ib640], rb0, gsem0)
        d1 = pltpu.async_copy(tab.at[ib128], rb1.at[pl.ds(0, 128)], gsem1)
        d0.wait()
        d1.wait()
        base = s * BPT

        @pl.when(c == 0)
        def _():
            pltpu.sync_copy(rb0, out_lo.at[layer, pl.ds(base, 640)])
            pltpu.sync_copy(rb1.at[pl.ds(0, 128)],
                            out_lo.at[layer, pl.ds(base + 640, 128)])

        @pl.when(c == 1)
        def _():
            pltpu.sync_copy(rb0, out_hi.at[layer, pl.ds(base, 640)])
            pltpu.sync_copy(rb1.at[pl.ds(0, 128)],
                            out_hi.at[layer, pl.ds(base + 640, 128)])

    def pipeline(tab, accr):
        _edge_pipeline(s, tab, accr, src_hbm, dst_hbm, w_hbm,
                       rb, sb, db, wb, esem, gsem, ssem)

    @pl.when(c == 0)
    def _():
        pltpu.async_copy(t_lo.at[pl.ds(r0, ROWS_PER_TILE)],
                         ta.at[pl.ds(r0, ROWS_PER_TILE)], ssem0)

    @pl.when(c == 1)
    def _():
        pltpu.async_copy(t_hi.at[pl.ds(r0, ROWS_PER_TILE)],
                         ta.at[pl.ds(r0, ROWS_PER_TILE)], ssem0)

    pltpu.sync_copy(idx_hbm.at[pl.ds(s * BPT, 640)], ib640)
    pltpu.sync_copy(idx_hbm.at[pl.ds(s * BPT + 640, 128)], ib128)
    zero_acc(tb)

    @pl.when(c == 0)
    def _():
        pltpu.make_async_copy(t_lo.at[pl.ds(r0, ROWS_PER_TILE)],
                              ta.at[pl.ds(r0, ROWS_PER_TILE)], ssem0).wait()

    @pl.when(c == 1)
    def _():
        pltpu.make_async_copy(t_hi.at[pl.ds(r0, ROWS_PER_TILE)],
                              ta.at[pl.ds(r0, ROWS_PER_TILE)], ssem0).wait()

    plsc.subcore_barrier()

    batch_gather(ta, 0)
    pipeline(ta, tb)
    batch_gather(tb, 1)
    zero_acc(ta)
    plsc.subcore_barrier()
    pipeline(tb, ta)
    batch_gather(ta, 2)
    zero_acc(tb)
    plsc.subcore_barrier()
    pipeline(ta, tb)
    batch_gather(tb, 3)


_mega = pl.kernel(
    _mega_body,
    out_type=(jax.ShapeDtypeStruct((4, B3, DH), jnp.float32),
              jax.ShapeDtypeStruct((4, B3, DH), jnp.float32)),
    mesh=_mesh,
    compiler_params=_sc_params,
    scratch_types=[
        pltpu.VMEM_SHARED((N_PAD, DH), jnp.float32),
        pltpu.VMEM_SHARED((N_PAD, DH), jnp.float32),
        pltpu.VMEM((CHUNK_ROWS * 128, DH), jnp.float32),
        pltpu.VMEM((CHUNK_ROWS * 128, DH), jnp.float32),
        pltpu.VMEM((CHUNK_ROWS * 128,), jnp.int32),
        pltpu.VMEM((CHUNK_ROWS * 128,), jnp.int32),
        pltpu.VMEM((CHUNK_ROWS * 128,), jnp.int32),
        pltpu.VMEM((CHUNK_ROWS * 128,), jnp.int32),
        pltpu.VMEM((CHUNK_ROWS * 128,), jnp.int32),
        pltpu.VMEM((CHUNK_ROWS * 128,), jnp.int32),
        pltpu.VMEM((CHUNK_ROWS * 128,), jnp.int32),
        pltpu.VMEM((CHUNK_ROWS * 128,), jnp.int32),
        pltpu.VMEM((CHUNK_ROWS * 128,), jnp.float32),
        pltpu.VMEM((CHUNK_ROWS * 128,), jnp.float32),
        pltpu.VMEM((CHUNK_ROWS * 128,), jnp.float32),
        pltpu.VMEM((CHUNK_ROWS * 128,), jnp.float32),
        pltpu.VMEM((640,), jnp.int32),
        pltpu.VMEM((128,), jnp.int32),
        pltpu.SemaphoreType.DMA,
        pltpu.SemaphoreType.DMA,
        pltpu.SemaphoreType.DMA,
        pltpu.SemaphoreType.DMA,
        pltpu.SemaphoreType.DMA,
        pltpu.SemaphoreType.DMA,
        pltpu.SemaphoreType.DMA,
        pltpu.SemaphoreType.DMA,
    ],
)


_BR = 1024


def _loss_body(ul, pl_, nl, uh, ph, nh, o_ref):
    i = pl.program_id(0)

    def avg(r):
        x = r[...]
        return (x[0] + x[1] + x[2] + x[3]) * 0.25

    ue = jnp.concatenate([avg(ul), avg(uh)], axis=1)
    pe = jnp.concatenate([avg(pl_), avg(ph)], axis=1)
    ne = jnp.concatenate([avg(nl), avg(nh)], axis=1)
    ps = jnp.sum(ue * pe, axis=1, keepdims=True)
    ns = jnp.sum(ue * ne, axis=1, keepdims=True)
    x = ns - ps
    sp = jnp.maximum(x, 0.0) + jnp.log1p(jnp.exp(-jnp.abs(x)))
    reg = (jnp.sum(ul[0] ** 2) + jnp.sum(uh[0] ** 2)
           + jnp.sum(pl_[0] ** 2) + jnp.sum(ph[0] ** 2)
           + jnp.sum(nl[0] ** 2) + jnp.sum(nh[0] ** 2))
    part = jnp.sum(sp) / float(BATCH) + (1e-4 * 0.5 / float(BATCH)) * reg

    @pl.when(i == 0)
    def _():
        o_ref[...] = jnp.zeros((1, 1), jnp.float32)

    o_ref[...] = o_ref[...] + jnp.full((1, 1), part, jnp.float32)


def _loss(blo, bhi):
    nb = BATCH // _BR
    bs = (4, _BR, DH)
    specs = []
    for off in (0, nb, 2 * nb):
        specs.append(pl.BlockSpec(bs, lambda i, off=off: (0, off + i, 0)))
    return pl.pallas_call(
        _loss_body,
        grid=(nb,),
        in_specs=[specs[0], specs[1], specs[2]] * 2,
        out_specs=pl.BlockSpec((1, 1), lambda i: (0, 0)),
        out_shape=jax.ShapeDtypeStruct((1, 1), jnp.float32),
    )(blo, blo, blo, bhi, bhi, bhi)


def kernel(user_emb, item_emb, edge_weight, edge_index, users, pos, neg):
    t0 = jnp.concatenate(
        [user_emb, item_emb,
         jnp.zeros((N_PAD - N_NODES, D), jnp.float32)], axis=0)
    t0l, t0h = t0[:, :DH], t0[:, DH:]
    pad = E_PAD - N_EDGES
    src = jnp.concatenate([edge_index[0], jnp.zeros((pad,), jnp.int32)])
    dst = jnp.concatenate([edge_index[1], jnp.zeros((pad,), jnp.int32)])
    w = jnp.concatenate([edge_weight, jnp.zeros((pad,), jnp.float32)])
    idx = jnp.concatenate([users, pos + N_USERS, neg + N_USERS])

    blo, bhi = _mega(t0l, t0h, src, dst, w, idx)
    out = _loss(blo, bhi)
    return out[0, 0]
